# baseline (speedup 1.0000x reference)
import jax
import jax.numpy as jnp
from jax import lax
from jax.experimental import pallas as pl
from jax.experimental.pallas import tpu as pltpu
from jax.experimental.pallas import tpu_sc as plsc

_B, _S, _D, _P = 4, 4096, 2048, 16
_BS = 1024
_NC, _NS, _L = 2, 16, 16


def _sc_gather_modify_body(x2_ref, rowidx_ref, pos_ref, chg_ref, out_ref,
                           rowidx_v, rows_v, pos_v, chg_v, final_v, sem):
    wid = lax.axis_index("s") * _NC + lax.axis_index("c")

    def _tile_work(b):
        pltpu.sync_copy(rowidx_ref, rowidx_v)
        idx16 = rowidx_v[...]
        lane = lax.broadcasted_iota(jnp.int32, (_L,), 0)
        pltpu.sync_copy(x2_ref.at[idx16[b]], rows_v.at[pl.ds(0, _D)])
        pltpu.sync_copy(pos_ref.at[b], pos_v)
        pltpu.sync_copy(chg_ref.at[b], chg_v)
        pos = pos_v[...]
        vals = jnp.zeros((_L,), jnp.float32)
        for p in range(_P):
            v_p = rows_v[pl.ds(pos[p], _L)][0]
            vals = jnp.where(lane == p, v_p, vals)
        delta = chg_v[...] * jnp.sign(vals)
        acc = vals
        for q in range(_L):
            pos_q = pos[q]
            d_q = delta[q]
            acc = acc + jnp.where(pos == pos_q, d_q, 0.0)
        final_v[...] = acc
        pltpu.sync_copy(final_v, out_ref.at[b])

    for _b in range(_B):
        pl.when(wid == _b)(lambda _b=_b: _tile_work(_b))


def _sc_gather_modify(x2, rowidx, pos, chg):
    mesh = plsc.VectorSubcoreMesh(core_axis_name="c", subcore_axis_name="s")
    import functools
    k = functools.partial(
        pl.kernel,
        mesh=mesh,
        out_type=jax.ShapeDtypeStruct((_B, _P), jnp.float32),
        scratch_types=[
            pltpu.VMEM((16,), jnp.int32),
            pltpu.VMEM((_D + _L,), jnp.float32),
            pltpu.VMEM((_P,), jnp.int32),
            pltpu.VMEM((_P,), jnp.float32),
            pltpu.VMEM((_P,), jnp.float32),
            pltpu.SemaphoreType.DMA,
        ],
    )(_sc_gather_modify_body)
    return k(x2, rowidx, pos, chg)


def _tc_copy_body(x_ref, o_ref):
    o_ref[0] = x_ref[0]


def _tc_apply_body(mask_sp, pos_ref, fin_ref, y_ref, o_ref):
    b = pl.program_id(0)
    r = mask_sp[b] % 8
    row = y_ref[0, pl.ds(r, 1), :]
    d_iota = lax.broadcasted_iota(jnp.int32, (1, _D), 1)
    for p in range(_P):
        row = jnp.where(d_iota == pos_ref[b, p], fin_ref[b, p], row)
    o_ref[0] = y_ref[0]
    o_ref[0, pl.ds(r, 1), :] = row


def kernel(x, mask_idxs, pos_positions, pos_changes):
    rowidx = jnp.arange(_B, dtype=jnp.int32) * _S + mask_idxs.astype(jnp.int32)
    rowidx = jnp.pad(rowidx, (0, 16 - _B))
    x2 = x.reshape(_B * _S, _D)
    finals = _sc_gather_modify(
        x2, rowidx, pos_positions.astype(jnp.int32), pos_changes
    )

    grid_spec = pltpu.PrefetchScalarGridSpec(
        num_scalar_prefetch=1,
        grid=(_B,),
        in_specs=[
            pl.BlockSpec(memory_space=pltpu.SMEM),
            pl.BlockSpec(memory_space=pltpu.SMEM),
            pl.BlockSpec((1, 8, _D), lambda b, mask_sp: (b, mask_sp[b] // 8, 0)),
        ],
        out_specs=pl.BlockSpec((1, 8, _D), lambda b, mask_sp: (b, mask_sp[b] // 8, 0)),
    )
    return pl.pallas_call(
        _tc_apply_body,
        grid_spec=grid_spec,
        out_shape=jax.ShapeDtypeStruct((_B, _S, _D), jnp.float32),
        input_output_aliases={3: 0},
    )(mask_idxs, pos_positions, finals, x)


# speedup vs baseline: 1.2823x; 1.2823x over previous
import jax
import jax.numpy as jnp
from jax import lax
from jax.experimental import pallas as pl
from jax.experimental.pallas import tpu as pltpu

_B, _S, _D, _P = 4, 4096, 2048, 16
_BS = 1024


def _patch_copy_body(mask_ref, pos_ref, chg_ref, x_ref, o_ref):
    blk = pl.program_id(0)
    o_ref[...] = x_ref[...]
    for b in range(_B):
        m = mask_ref[b] + b * _S

        @pl.when(m // _BS == blk)
        def _patch():
            r = m - blk * _BS
            row = x_ref[pl.ds(r, 1), :]
            d_iota = lax.broadcasted_iota(jnp.int32, (1, _D), 1)
            delta_row = jnp.zeros((1, _D), jnp.float32)
            for p in range(_P):
                pos_p = pos_ref[b, p]
                onehot = d_iota == pos_p
                val_p = jnp.sum(jnp.where(onehot, row, 0.0))
                delta_row = delta_row + jnp.where(
                    onehot, chg_ref[b, p] * jnp.sign(val_p), 0.0
                )
            o_ref[pl.ds(r, 1), :] = row + delta_row


def kernel(x, mask_idxs, pos_positions, pos_changes):
    x2 = x.reshape(_B * _S, _D)
    out = pl.pallas_call(
        _patch_copy_body,
        grid=(_B * _S // _BS,),
        in_specs=[
            pl.BlockSpec(memory_space=pltpu.SMEM),
            pl.BlockSpec(memory_space=pltpu.SMEM),
            pl.BlockSpec(memory_space=pltpu.SMEM),
            pl.BlockSpec((_BS, _D), lambda i: (i, 0)),
        ],
        out_specs=pl.BlockSpec((_BS, _D), lambda i: (i, 0)),
        out_shape=jax.ShapeDtypeStruct((_B * _S, _D), jnp.float32),
        compiler_params=pltpu.CompilerParams(
            dimension_semantics=("parallel",),
        ),
    )(mask_idxs, pos_positions, pos_changes, x2)
    return out.reshape(_B, _S, _D)


# speedup vs baseline: 1.2863x; 1.0031x over previous
import jax
import jax.numpy as jnp
from jax import lax
from jax.experimental import pallas as pl
from jax.experimental.pallas import tpu as pltpu

_B, _S, _D, _P = 4, 4096, 2048, 16
_BS = 1024


def _patch_copy_body(mask_ref, pos_ref, chg_ref, x_ref, o_ref):
    b = pl.program_id(0)
    sblk = pl.program_id(1)
    m = mask_ref[b]
    o_ref[0] = x_ref[0]

    @pl.when(m // _BS == sblk)
    def _patch():
        r = m - sblk * _BS
        row = x_ref[0, pl.ds(r, 1), :]
        d_iota = lax.broadcasted_iota(jnp.int32, (1, _D), 1)
        delta_row = jnp.zeros((1, _D), jnp.float32)
        for p in range(_P):
            pos_p = pos_ref[b, p]
            onehot = d_iota == pos_p
            val_p = jnp.sum(jnp.where(onehot, row, 0.0))
            delta_row = delta_row + jnp.where(
                onehot, chg_ref[b, p] * jnp.sign(val_p), 0.0
            )
        o_ref[0, pl.ds(r, 1), :] = row + delta_row


def kernel(x, mask_idxs, pos_positions, pos_changes):
    grid = (_B, _S // _BS)
    return pl.pallas_call(
        _patch_copy_body,
        grid=grid,
        in_specs=[
            pl.BlockSpec(memory_space=pltpu.SMEM),
            pl.BlockSpec(memory_space=pltpu.SMEM),
            pl.BlockSpec(memory_space=pltpu.SMEM),
            pl.BlockSpec((1, _BS, _D), lambda b, s: (b, s, 0)),
        ],
        out_specs=pl.BlockSpec((1, _BS, _D), lambda b, s: (b, s, 0)),
        out_shape=jax.ShapeDtypeStruct((_B, _S, _D), jnp.float32),
        compiler_params=pltpu.CompilerParams(
            dimension_semantics=("parallel", "parallel"),
        ),
    )(mask_idxs, pos_positions, pos_changes, x)
